# Initial kernel scaffold; baseline (speedup 1.0000x reference)
import functools

import jax
import jax.numpy as jnp
from jax import lax
from jax.experimental import pallas as pl
from jax.experimental.pallas import tpu as pltpu
from jax.experimental.pallas import tpu_sc as plsc

N = 10000
E = 320000
D = 128
NSUB = 16
CHUNK = 128
CHUNKS_PER_TILE = 157
EPT = CHUNK * CHUNKS_PER_TILE
EPAD = EPT * NSUB
PAD = EPAD - E
ROWS_PER_TILE = 640
ACC_ROWS = ROWS_PER_TILE * NSUB

BLK = 1000
NB = N // BLK


def _sc_body(feat, shuf, srcg, dstg, srcd, dstd, zblk,
             s1, s2, s3, s4,
             acc, src_v, dst_v, rows_v, zblk_v, sem):
    s = lax.axis_index("s")
    c = lax.axis_index("c")
    pltpu.sync_copy(zblk, zblk_v)

    def run(table, src_hbm, dst_hbm, out_hbm):
        for k in range(ROWS_PER_TILE // CHUNK):
            pltpu.sync_copy(
                zblk_v, acc.at[pl.ds(s * ROWS_PER_TILE + k * CHUNK, CHUNK)])
        pltpu.sync_copy(src_hbm.at[s], src_v)
        pltpu.sync_copy(dst_hbm.at[s], dst_v)
        plsc.subcore_barrier()

        def body(j, carry):
            pltpu.async_copy(table.at[src_v.at[j]], rows_v, sem).wait()
            pltpu.sync_copy(rows_v, acc.at[dst_v.at[j]], add=True)
            return carry

        lax.fori_loop(0, CHUNKS_PER_TILE, body, 0)
        plsc.subcore_barrier()
        pltpu.sync_copy(acc.at[pl.ds(s * ROWS_PER_TILE, ROWS_PER_TILE)],
                        out_hbm.at[pl.ds(s * ROWS_PER_TILE, ROWS_PER_TILE)])
        plsc.subcore_barrier()

    @pl.when(c == 0)
    def _():
        run(feat, srcg, dstg, s1)
        run(feat, srcd, dstd, s2)

    @pl.when(c == 1)
    def _():
        run(shuf, srcg, dstg, s3)
        run(shuf, srcd, dstd, s4)


_sc_seg = functools.partial(
    pl.kernel,
    out_type=[jax.ShapeDtypeStruct((ACC_ROWS, D), jnp.float32)] * 4,
    mesh=plsc.VectorSubcoreMesh(core_axis_name="c", subcore_axis_name="s"),
    scratch_types=[
        pltpu.VMEM_SHARED((ACC_ROWS, D), jnp.float32),
        pltpu.VMEM((CHUNKS_PER_TILE, CHUNK), jnp.int32),
        pltpu.VMEM((CHUNKS_PER_TILE, CHUNK), jnp.int32),
        pltpu.VMEM((CHUNK, D), jnp.float32),
        pltpu.VMEM((CHUNK, D), jnp.float32),
        pltpu.SemaphoreType.DMA,
    ],
)(_sc_body)


def _prep_edges(ei):
    r = jnp.arange(PAD, dtype=jnp.int32)
    src = jnp.concatenate([ei[0], r % N])
    dst = jnp.concatenate([ei[1], N + r % (ACC_ROWS - N)])
    return (src.reshape(NSUB, CHUNKS_PER_TILE, CHUNK),
            dst.reshape(NSUB, CHUNKS_PER_TILE, CHUNK))


def _tc_body(s1, s2, s3, s4, w1, w2, b1, b2,
             h1, h2, c1, c2, h3, h4, acc1, acc2):
    i = pl.program_id(0)
    a1 = jnp.dot(s1[...], w1[...], preferred_element_type=jnp.float32) + b1[...]
    a2 = jnp.dot(s2[...], w2[...], preferred_element_type=jnp.float32) + b2[...]
    a3 = jnp.dot(s3[...], w1[...], preferred_element_type=jnp.float32) + b1[...]
    a4 = jnp.dot(s4[...], w2[...], preferred_element_type=jnp.float32) + b2[...]
    h1[...] = a1
    h2[...] = a2
    h3[...] = a3
    h4[...] = a4
    p1 = jnp.sum(a1, axis=0, keepdims=True)
    p2 = jnp.sum(a2, axis=0, keepdims=True)

    @pl.when(i == 0)
    def _():
        acc1[...] = p1
        acc2[...] = p2

    @pl.when(i > 0)
    def _():
        acc1[...] += p1
        acc2[...] += p2

    @pl.when(i == NB - 1)
    def _():
        c1[...] = jax.nn.sigmoid(acc1[...] / N)
        c2[...] = jax.nn.sigmoid(acc2[...] / N)


def _tc_head(s1, s2, s3, s4, w1, w2, b1, b2):
    row = pl.BlockSpec((BLK, D), lambda i: (i, 0))
    full = pl.BlockSpec((D, D), lambda i: (0, 0))
    vec = pl.BlockSpec((1, D), lambda i: (0, 0))
    return pl.pallas_call(
        _tc_body,
        grid=(NB,),
        in_specs=[row, row, row, row, full, full, vec, vec],
        out_specs=[row, row, vec, vec, row, row],
        out_shape=[
            jax.ShapeDtypeStruct((N, D), jnp.float32),
            jax.ShapeDtypeStruct((N, D), jnp.float32),
            jax.ShapeDtypeStruct((1, D), jnp.float32),
            jax.ShapeDtypeStruct((1, D), jnp.float32),
            jax.ShapeDtypeStruct((N, D), jnp.float32),
            jax.ShapeDtypeStruct((N, D), jnp.float32),
        ],
        scratch_shapes=[
            pltpu.VMEM((1, D), jnp.float32),
            pltpu.VMEM((1, D), jnp.float32),
        ],
    )(s1, s2, s3, s4, w1, w2, b1, b2)


def kernel(feat, shuf_feat, graph_edge_index, dif_edge_index, W1, b1, W2, b2):
    srcg, dstg = _prep_edges(graph_edge_index)
    srcd, dstd = _prep_edges(dif_edge_index)
    zblk = jnp.zeros((CHUNK, D), jnp.float32)
    s1, s2, s3, s4 = _sc_seg(feat, shuf_feat, srcg, dstg, srcd, dstd, zblk)
    h1, h2, c1, c2, h3, h4 = _tc_head(
        s1, s2, s3, s4, W1, W2, b1.reshape(1, D), b2.reshape(1, D))
    return h1, h2, c1, c2, h3, h4


# speedup vs baseline: 5.4641x; 5.4641x over previous
import functools

import jax
import jax.numpy as jnp
from jax import lax
from jax.experimental import pallas as pl
from jax.experimental.pallas import tpu as pltpu
from jax.experimental.pallas import tpu_sc as plsc

N = 10000
E = 320000
D = 128
NSUB = 16
CHUNK = 128
GROUP = 32
NGROUP = 5
CHUNKS_PER_TILE = GROUP * NGROUP
EPT = CHUNK * CHUNKS_PER_TILE
EPAD = EPT * NSUB
PAD = EPAD - E
ROWS_PER_TILE = 640
ACC_ROWS = ROWS_PER_TILE * NSUB

BLK = 1000
NB = N // BLK


def _sc_body(feat, shuf, srcg, dstg, srcd, dstd, zblk,
             s1, s2, s3, s4,
             acc, src_v, dst_v, rows_v, sem):
    s = lax.axis_index("s")
    c = lax.axis_index("c")

    def run(table, src_hbm, dst_hbm, out_hbm):
        for k in range(ROWS_PER_TILE // CHUNK):
            pltpu.sync_copy(
                zblk, acc.at[pl.ds(s * ROWS_PER_TILE + k * CHUNK, CHUNK)])
        plsc.subcore_barrier()

        def group(g, carry):
            pltpu.sync_copy(src_hbm.at[s, g], src_v)
            pltpu.sync_copy(dst_hbm.at[s, g], dst_v)

            def body(j, carry2):
                pltpu.async_copy(table.at[src_v.at[j]], rows_v, sem).wait()
                pltpu.sync_copy(rows_v, acc.at[dst_v.at[j]], add=True)
                return carry2

            return lax.fori_loop(0, GROUP, body, carry)

        lax.fori_loop(0, NGROUP, group, 0)
        plsc.subcore_barrier()
        pltpu.sync_copy(acc.at[pl.ds(s * ROWS_PER_TILE, ROWS_PER_TILE)],
                        out_hbm.at[pl.ds(s * ROWS_PER_TILE, ROWS_PER_TILE)])
        plsc.subcore_barrier()

    @pl.when(c == 0)
    def _():
        run(feat, srcg, dstg, s1)
        run(feat, srcd, dstd, s2)

    @pl.when(c == 1)
    def _():
        run(shuf, srcg, dstg, s3)
        run(shuf, srcd, dstd, s4)


_sc_seg = functools.partial(
    pl.kernel,
    out_type=[jax.ShapeDtypeStruct((ACC_ROWS, D), jnp.float32)] * 4,
    mesh=plsc.VectorSubcoreMesh(core_axis_name="c", subcore_axis_name="s"),
    scratch_types=[
        pltpu.VMEM_SHARED((ACC_ROWS, D), jnp.float32),
        pltpu.VMEM((GROUP, CHUNK), jnp.int32),
        pltpu.VMEM((GROUP, CHUNK), jnp.int32),
        pltpu.VMEM((CHUNK, D), jnp.float32),
        pltpu.SemaphoreType.DMA,
    ],
)(_sc_body)


def _prep_edges(ei):
    r = jnp.arange(PAD, dtype=jnp.int32)
    src = jnp.concatenate([ei[0], r % N])
    dst = jnp.concatenate([ei[1], N + r % (ACC_ROWS - N)])
    return (src.reshape(NSUB, NGROUP, GROUP, CHUNK),
            dst.reshape(NSUB, NGROUP, GROUP, CHUNK))


def _tc_body(s1, s2, s3, s4, w1, w2, b1, b2,
             h1, h2, c1, c2, h3, h4, acc1, acc2):
    i = pl.program_id(0)
    a1 = jnp.dot(s1[...], w1[...], preferred_element_type=jnp.float32) + b1[...]
    a2 = jnp.dot(s2[...], w2[...], preferred_element_type=jnp.float32) + b2[...]
    a3 = jnp.dot(s3[...], w1[...], preferred_element_type=jnp.float32) + b1[...]
    a4 = jnp.dot(s4[...], w2[...], preferred_element_type=jnp.float32) + b2[...]
    h1[...] = a1
    h2[...] = a2
    h3[...] = a3
    h4[...] = a4
    p1 = jnp.sum(a1, axis=0, keepdims=True)
    p2 = jnp.sum(a2, axis=0, keepdims=True)

    @pl.when(i == 0)
    def _():
        acc1[...] = p1
        acc2[...] = p2

    @pl.when(i > 0)
    def _():
        acc1[...] += p1
        acc2[...] += p2

    @pl.when(i == NB - 1)
    def _():
        c1[...] = jax.nn.sigmoid(acc1[...] / N)
        c2[...] = jax.nn.sigmoid(acc2[...] / N)


def _tc_head(s1, s2, s3, s4, w1, w2, b1, b2):
    row = pl.BlockSpec((BLK, D), lambda i: (i, 0))
    full = pl.BlockSpec((D, D), lambda i: (0, 0))
    vec = pl.BlockSpec((1, D), lambda i: (0, 0))
    return pl.pallas_call(
        _tc_body,
        grid=(NB,),
        in_specs=[row, row, row, row, full, full, vec, vec],
        out_specs=[row, row, vec, vec, row, row],
        out_shape=[
            jax.ShapeDtypeStruct((N, D), jnp.float32),
            jax.ShapeDtypeStruct((N, D), jnp.float32),
            jax.ShapeDtypeStruct((1, D), jnp.float32),
            jax.ShapeDtypeStruct((1, D), jnp.float32),
            jax.ShapeDtypeStruct((N, D), jnp.float32),
            jax.ShapeDtypeStruct((N, D), jnp.float32),
        ],
        scratch_shapes=[
            pltpu.VMEM((1, D), jnp.float32),
            pltpu.VMEM((1, D), jnp.float32),
        ],
    )(s1, s2, s3, s4, w1, w2, b1, b2)


def kernel(feat, shuf_feat, graph_edge_index, dif_edge_index, W1, b1, W2, b2):
    srcg, dstg = _prep_edges(graph_edge_index)
    srcd, dstd = _prep_edges(dif_edge_index)
    zblk = jnp.zeros((CHUNK, D), jnp.float32)
    s1, s2, s3, s4 = _sc_seg(feat, shuf_feat, srcg, dstg, srcd, dstd, zblk)
    h1, h2, c1, c2, h3, h4 = _tc_head(
        s1, s2, s3, s4, W1, W2, b1.reshape(1, D), b2.reshape(1, D))
    return h1, h2, c1, c2, h3, h4


# speedup vs baseline: 7.1723x; 1.3126x over previous
import functools

import jax
import jax.numpy as jnp
from jax import lax
from jax.experimental import pallas as pl
from jax.experimental.pallas import tpu as pltpu
from jax.experimental.pallas import tpu_sc as plsc

N = 10000
E = 320000
D = 128
NSUB = 16
CHUNK = 128
GROUP = 32
NGROUP = 5
CHUNKS_PER_TILE = GROUP * NGROUP
EPT = CHUNK * CHUNKS_PER_TILE
EPAD = EPT * NSUB
PAD = EPAD - E
ROWS_PER_TILE = 640
ACC_ROWS = ROWS_PER_TILE * NSUB

BLK = 1000
NB = N // BLK


def _sc_body(feat, shuf, srcg, dstg, srcd, dstd, zblk,
             s1, s2, s3, s4,
             acc, src_v, dst_v, rows_a, rows_b, sem_a, sem_b):
    s = lax.axis_index("s")
    c = lax.axis_index("c")
    NPAIR = GROUP // 2

    def run(table, src_hbm, dst_hbm, out_hbm):
        def g_start(j, buf, sem):
            pltpu.async_copy(table.at[src_v.at[j]], buf, sem)

        def g_wait(buf, sem):
            pltpu.make_async_copy(table.at[pl.ds(0, CHUNK)], buf, sem).wait()

        def scat(j, buf):
            pltpu.sync_copy(buf, acc.at[dst_v.at[j]], add=True)

        for k in range(ROWS_PER_TILE // CHUNK):
            pltpu.sync_copy(
                zblk, acc.at[pl.ds(s * ROWS_PER_TILE + k * CHUNK, CHUNK)])
        plsc.subcore_barrier()

        def group(g, carry):
            pltpu.sync_copy(src_hbm.at[s, g], src_v)
            pltpu.sync_copy(dst_hbm.at[s, g], dst_v)
            g_start(0, rows_a, sem_a)

            def pair(p, carry2):
                g_wait(rows_a, sem_a)
                g_start(2 * p + 1, rows_b, sem_b)
                scat(2 * p, rows_a)
                g_wait(rows_b, sem_b)

                @pl.when(p < NPAIR - 1)
                def _():
                    g_start(2 * p + 2, rows_a, sem_a)

                scat(2 * p + 1, rows_b)
                return carry2

            return lax.fori_loop(0, NPAIR, pair, carry)

        lax.fori_loop(0, NGROUP, group, 0)
        plsc.subcore_barrier()
        pltpu.sync_copy(acc.at[pl.ds(s * ROWS_PER_TILE, ROWS_PER_TILE)],
                        out_hbm.at[pl.ds(s * ROWS_PER_TILE, ROWS_PER_TILE)])
        plsc.subcore_barrier()

    @pl.when(c == 0)
    def _():
        run(feat, srcg, dstg, s1)
        run(feat, srcd, dstd, s2)

    @pl.when(c == 1)
    def _():
        run(shuf, srcg, dstg, s3)
        run(shuf, srcd, dstd, s4)


_sc_seg = functools.partial(
    pl.kernel,
    out_type=[jax.ShapeDtypeStruct((ACC_ROWS, D), jnp.float32)] * 4,
    mesh=plsc.VectorSubcoreMesh(core_axis_name="c", subcore_axis_name="s"),
    scratch_types=[
        pltpu.VMEM_SHARED((ACC_ROWS, D), jnp.float32),
        pltpu.VMEM((GROUP, CHUNK), jnp.int32),
        pltpu.VMEM((GROUP, CHUNK), jnp.int32),
        pltpu.VMEM((CHUNK, D), jnp.float32),
        pltpu.VMEM((CHUNK, D), jnp.float32),
        pltpu.SemaphoreType.DMA,
        pltpu.SemaphoreType.DMA,
    ],
)(_sc_body)


def _prep_edges(ei):
    r = jnp.arange(PAD, dtype=jnp.int32)
    src = jnp.concatenate([ei[0], r % N])
    dst = jnp.concatenate([ei[1], N + r % (ACC_ROWS - N)])
    return (src.reshape(NSUB, NGROUP, GROUP, CHUNK),
            dst.reshape(NSUB, NGROUP, GROUP, CHUNK))


def _tc_body(s1, s2, s3, s4, w1, w2, b1, b2,
             h1, h2, c1, c2, h3, h4, acc1, acc2):
    i = pl.program_id(0)
    a1 = jnp.dot(s1[...], w1[...], preferred_element_type=jnp.float32) + b1[...]
    a2 = jnp.dot(s2[...], w2[...], preferred_element_type=jnp.float32) + b2[...]
    a3 = jnp.dot(s3[...], w1[...], preferred_element_type=jnp.float32) + b1[...]
    a4 = jnp.dot(s4[...], w2[...], preferred_element_type=jnp.float32) + b2[...]
    h1[...] = a1
    h2[...] = a2
    h3[...] = a3
    h4[...] = a4
    p1 = jnp.sum(a1, axis=0, keepdims=True)
    p2 = jnp.sum(a2, axis=0, keepdims=True)

    @pl.when(i == 0)
    def _():
        acc1[...] = p1
        acc2[...] = p2

    @pl.when(i > 0)
    def _():
        acc1[...] += p1
        acc2[...] += p2

    @pl.when(i == NB - 1)
    def _():
        c1[...] = jax.nn.sigmoid(acc1[...] / N)
        c2[...] = jax.nn.sigmoid(acc2[...] / N)


def _tc_head(s1, s2, s3, s4, w1, w2, b1, b2):
    row = pl.BlockSpec((BLK, D), lambda i: (i, 0))
    full = pl.BlockSpec((D, D), lambda i: (0, 0))
    vec = pl.BlockSpec((1, D), lambda i: (0, 0))
    return pl.pallas_call(
        _tc_body,
        grid=(NB,),
        in_specs=[row, row, row, row, full, full, vec, vec],
        out_specs=[row, row, vec, vec, row, row],
        out_shape=[
            jax.ShapeDtypeStruct((N, D), jnp.float32),
            jax.ShapeDtypeStruct((N, D), jnp.float32),
            jax.ShapeDtypeStruct((1, D), jnp.float32),
            jax.ShapeDtypeStruct((1, D), jnp.float32),
            jax.ShapeDtypeStruct((N, D), jnp.float32),
            jax.ShapeDtypeStruct((N, D), jnp.float32),
        ],
        scratch_shapes=[
            pltpu.VMEM((1, D), jnp.float32),
            pltpu.VMEM((1, D), jnp.float32),
        ],
    )(s1, s2, s3, s4, w1, w2, b1, b2)


def kernel(feat, shuf_feat, graph_edge_index, dif_edge_index, W1, b1, W2, b2):
    srcg, dstg = _prep_edges(graph_edge_index)
    srcd, dstd = _prep_edges(dif_edge_index)
    zblk = jnp.zeros((CHUNK, D), jnp.float32)
    s1, s2, s3, s4 = _sc_seg(feat, shuf_feat, srcg, dstg, srcd, dstd, zblk)
    h1, h2, c1, c2, h3, h4 = _tc_head(
        s1, s2, s3, s4, W1, W2, b1.reshape(1, D), b2.reshape(1, D))
    return h1, h2, c1, c2, h3, h4


# speedup vs baseline: 7.4781x; 1.0426x over previous
import functools

import jax
import jax.numpy as jnp
from jax import lax
from jax.experimental import pallas as pl
from jax.experimental.pallas import tpu as pltpu
from jax.experimental.pallas import tpu_sc as plsc

N = 10000
E = 320000
D = 128
NSUB = 16
CHUNK = 64
GROUP = 64
NGROUP = 5
CHUNKS_PER_TILE = GROUP * NGROUP
EPT = CHUNK * CHUNKS_PER_TILE
EPAD = EPT * NSUB
PAD = EPAD - E
ROWS_PER_TILE = 640
ACC_ROWS = ROWS_PER_TILE * NSUB

BLK = 1000
NB = N // BLK


def _sc_body(feat, shuf, srcg, dstg, srcd, dstd, zblk,
             s1, s2, s3, s4,
             acc, src_v, dst_v, r0, r1, r2, r3,
             sg0, sg1, sg2, sg3, ss0, ss1, ss2, ss3):
    s = lax.axis_index("s")
    c = lax.axis_index("c")
    bufs = (r0, r1, r2, r3)
    gsems = (sg0, sg1, sg2, sg3)
    ssems = (ss0, ss1, ss2, ss3)

    def run(table, src_hbm, dst_hbm, out_hbm):
        def g_start(j, b):
            pltpu.async_copy(table.at[src_v.at[j]], bufs[b], gsems[b])

        def g_wait(b):
            pltpu.make_async_copy(table.at[pl.ds(0, CHUNK)],
                                  bufs[b], gsems[b]).wait()

        def s_start(j, b):
            pltpu.async_copy(bufs[b], acc.at[dst_v.at[j]], ssems[b], add=True)

        def s_wait(b):
            pltpu.make_async_copy(bufs[b], acc.at[pl.ds(0, CHUNK)],
                                  ssems[b]).wait()

        for k in range(ROWS_PER_TILE // CHUNK):
            pltpu.sync_copy(
                zblk, acc.at[pl.ds(s * ROWS_PER_TILE + k * CHUNK, CHUNK)])
        plsc.subcore_barrier()

        def group(g, carry):
            pltpu.sync_copy(src_hbm.at[s, g], src_v)
            pltpu.sync_copy(dst_hbm.at[s, g], dst_v)
            g_start(0, 0)
            g_start(1, 1)
            for j in (0, 1):
                g_wait(j)
                s_start(j, j)
                g_start(j + 2, (j + 2) % 4)

            def quad(q, carry2):
                j0 = 2 + 4 * q
                for bb in range(4):
                    j = j0 + bb
                    b = (2 + bb) % 4
                    g_wait(b)
                    s_start(j, b)
                    s_wait(bb)
                    g_start(j + 2, bb)
                return carry2

            lax.fori_loop(0, (GROUP - 4) // 4, quad, carry)
            for j in (GROUP - 2, GROUP - 1):
                b = j % 4
                g_wait(b)
                s_start(j, b)
                s_wait((j + 2) % 4)
            s_wait((GROUP - 2) % 4)
            s_wait((GROUP - 1) % 4)
            return carry

        lax.fori_loop(0, NGROUP, group, 0)
        plsc.subcore_barrier()
        pltpu.sync_copy(acc.at[pl.ds(s * ROWS_PER_TILE, ROWS_PER_TILE)],
                        out_hbm.at[pl.ds(s * ROWS_PER_TILE, ROWS_PER_TILE)])
        plsc.subcore_barrier()

    @pl.when(c == 0)
    def _():
        run(feat, srcg, dstg, s1)
        run(feat, srcd, dstd, s2)

    @pl.when(c == 1)
    def _():
        run(shuf, srcg, dstg, s3)
        run(shuf, srcd, dstd, s4)


_sc_seg = functools.partial(
    pl.kernel,
    out_type=[jax.ShapeDtypeStruct((ACC_ROWS, D), jnp.float32)] * 4,
    mesh=plsc.VectorSubcoreMesh(core_axis_name="c", subcore_axis_name="s"),
    scratch_types=[
        pltpu.VMEM_SHARED((ACC_ROWS, D), jnp.float32),
        pltpu.VMEM((GROUP, CHUNK), jnp.int32),
        pltpu.VMEM((GROUP, CHUNK), jnp.int32),
        pltpu.VMEM((CHUNK, D), jnp.float32),
        pltpu.VMEM((CHUNK, D), jnp.float32),
        pltpu.VMEM((CHUNK, D), jnp.float32),
        pltpu.VMEM((CHUNK, D), jnp.float32),
        pltpu.SemaphoreType.DMA,
        pltpu.SemaphoreType.DMA,
        pltpu.SemaphoreType.DMA,
        pltpu.SemaphoreType.DMA,
        pltpu.SemaphoreType.DMA,
        pltpu.SemaphoreType.DMA,
        pltpu.SemaphoreType.DMA,
        pltpu.SemaphoreType.DMA,
    ],
)(_sc_body)


def _prep_edges(ei):
    r = jnp.arange(PAD, dtype=jnp.int32)
    src = jnp.concatenate([ei[0], r % N])
    dst = jnp.concatenate([ei[1], N + r % (ACC_ROWS - N)])
    return (src.reshape(NSUB, NGROUP, GROUP, CHUNK),
            dst.reshape(NSUB, NGROUP, GROUP, CHUNK))


def _tc_body(s1, s2, s3, s4, w1, w2, b1, b2,
             h1, h2, c1, c2, h3, h4, acc1, acc2):
    i = pl.program_id(0)
    a1 = jnp.dot(s1[...], w1[...], preferred_element_type=jnp.float32) + b1[...]
    a2 = jnp.dot(s2[...], w2[...], preferred_element_type=jnp.float32) + b2[...]
    a3 = jnp.dot(s3[...], w1[...], preferred_element_type=jnp.float32) + b1[...]
    a4 = jnp.dot(s4[...], w2[...], preferred_element_type=jnp.float32) + b2[...]
    h1[...] = a1
    h2[...] = a2
    h3[...] = a3
    h4[...] = a4
    p1 = jnp.sum(a1, axis=0, keepdims=True)
    p2 = jnp.sum(a2, axis=0, keepdims=True)

    @pl.when(i == 0)
    def _():
        acc1[...] = p1
        acc2[...] = p2

    @pl.when(i > 0)
    def _():
        acc1[...] += p1
        acc2[...] += p2

    @pl.when(i == NB - 1)
    def _():
        c1[...] = jax.nn.sigmoid(acc1[...] / N)
        c2[...] = jax.nn.sigmoid(acc2[...] / N)


def _tc_head(s1, s2, s3, s4, w1, w2, b1, b2):
    row = pl.BlockSpec((BLK, D), lambda i: (i, 0))
    full = pl.BlockSpec((D, D), lambda i: (0, 0))
    vec = pl.BlockSpec((1, D), lambda i: (0, 0))
    return pl.pallas_call(
        _tc_body,
        grid=(NB,),
        in_specs=[row, row, row, row, full, full, vec, vec],
        out_specs=[row, row, vec, vec, row, row],
        out_shape=[
            jax.ShapeDtypeStruct((N, D), jnp.float32),
            jax.ShapeDtypeStruct((N, D), jnp.float32),
            jax.ShapeDtypeStruct((1, D), jnp.float32),
            jax.ShapeDtypeStruct((1, D), jnp.float32),
            jax.ShapeDtypeStruct((N, D), jnp.float32),
            jax.ShapeDtypeStruct((N, D), jnp.float32),
        ],
        scratch_shapes=[
            pltpu.VMEM((1, D), jnp.float32),
            pltpu.VMEM((1, D), jnp.float32),
        ],
    )(s1, s2, s3, s4, w1, w2, b1, b2)


def kernel(feat, shuf_feat, graph_edge_index, dif_edge_index, W1, b1, W2, b2):
    srcg, dstg = _prep_edges(graph_edge_index)
    srcd, dstd = _prep_edges(dif_edge_index)
    zblk = jnp.zeros((CHUNK, D), jnp.float32)
    s1, s2, s3, s4 = _sc_seg(feat, shuf_feat, srcg, dstg, srcd, dstd, zblk)
    h1, h2, c1, c2, h3, h4 = _tc_head(
        s1, s2, s3, s4, W1, W2, b1.reshape(1, D), b2.reshape(1, D))
    return h1, h2, c1, c2, h3, h4
